# Initial kernel scaffold; baseline (speedup 1.0000x reference)
import jax
import jax.numpy as jnp
from jax.experimental import pallas as pl


def kernel(x, ji_pairs, e_ji, e_ji_basis, Wf1, bf1, Wf2, bf2, Wl1, Wl2, bl2, Wl3, bl3):
    raise NotImplementedError("write your pallas kernel here")


# speedup vs baseline: 2.1931x; 2.1931x over previous
import functools
import math

import jax
import jax.numpy as jnp
from jax import lax
from jax.experimental import pallas as pl
from jax.experimental.pallas import tpu as pltpu
from jax.experimental.pallas import tpu_sc as plsc

_N = 10000
_E = 320000
_H = 128
_G = 50
_F = 128
_CUTOFF = 10.0

_BE = 2560
_NBLK = _E // _BE

_NC = 2
_NS = 16
_NW = _NC * _NS
_EW = _E // _NW
_CH = 80
_NCH = _EW // _CH
_ACC = 10240
_STRIPE = _ACC // _NS

_LOG2 = math.log(2.0)


def _ssp(v):
    return jnp.maximum(v, 0.0) + jnp.log1p(jnp.exp(-jnp.abs(v))) - _LOG2


def _dot_t(a, b):
    return lax.dot_general(a, b, (((1,), (1,)), ((), ())),
                           preferred_element_type=jnp.float32)


def _filter_body(basis_ref, e_ref, wf1_ref, bf1_ref, wf2_ref, bf2_ref,
                 w_ref, c_ref):
    h = _ssp(_dot_t(basis_ref[...], wf1_ref[...]) + bf1_ref[...])
    w_ref[...] = _dot_t(h, wf2_ref[...]) + bf2_ref[...]
    c_ref[...] = 0.25 * (jnp.cos(e_ref[...] * (math.pi / _CUTOFF)) + 1.0)


_filter_call = pl.pallas_call(
    _filter_body,
    grid=(_NBLK,),
    in_specs=[
        pl.BlockSpec((_BE, _G), lambda i: (i, 0)),
        pl.BlockSpec((1, 1, _BE), lambda i: (i, 0, 0)),
        pl.BlockSpec((_F, _G), lambda i: (0, 0)),
        pl.BlockSpec((1, _F), lambda i: (0, 0)),
        pl.BlockSpec((_F, _F), lambda i: (0, 0)),
        pl.BlockSpec((1, _F), lambda i: (0, 0)),
    ],
    out_specs=[
        pl.BlockSpec((_BE, _F), lambda i: (i, 0)),
        pl.BlockSpec((1, 1, _BE), lambda i: (i, 0, 0)),
    ],
    out_shape=[
        jax.ShapeDtypeStruct((_E, _F), jnp.float32),
        jax.ShapeDtypeStruct((_NBLK, 1, _BE), jnp.float32),
    ],
)


def _lin1_body(x_ref, wl1_ref, o_ref):
    o_ref[...] = _dot_t(x_ref[...], wl1_ref[...])


_lin1_call = pl.pallas_call(
    _lin1_body,
    grid=(5,),
    in_specs=[
        pl.BlockSpec((_N // 5, _H), lambda i: (i, 0)),
        pl.BlockSpec((_F, _H), lambda i: (0, 0)),
    ],
    out_specs=pl.BlockSpec((_N // 5, _F), lambda i: (i, 0)),
    out_shape=jax.ShapeDtypeStruct((_N, _F), jnp.float32),
)


_sc_mesh = plsc.VectorSubcoreMesh(core_axis_name="c", subcore_axis_name="s")


@functools.partial(
    pl.kernel,
    mesh=_sc_mesh,
    out_type=jax.ShapeDtypeStruct((_NC, _ACC, _H), jnp.float32),
    scratch_types=[
        pltpu.VMEM((_CH,), jnp.int32),
        pltpu.VMEM((_CH,), jnp.int32),
        pltpu.VMEM((_CH,), jnp.float32),
        pltpu.VMEM((_CH, _H), jnp.float32),
        pltpu.VMEM((_CH, _H), jnp.float32),
        pltpu.VMEM_SHARED((_ACC, _H), jnp.float32),
        pltpu.SemaphoreType.DMA,
    ],
)
def _sc_aggregate(x1_hbm, w_hbm, c_hbm, src_hbm, dst_hbm, out_hbm,
                  sidx, didx, cbuf, wbuf, rbuf, acc, sem):
    cid = lax.axis_index("c")
    sid = lax.axis_index("s")
    wid = cid * _NS + sid

    def _zrow(e, carry):
        for j in range(_H // 16):
            rbuf[e, pl.ds(j * 16, 16)] = jnp.zeros((16,), jnp.float32)
        return carry

    lax.fori_loop(0, _CH, _zrow, 0)

    def _zcopy(k, carry):
        pltpu.sync_copy(rbuf, acc.at[pl.ds(sid * _STRIPE + k * _CH, _CH)])
        return carry

    lax.fori_loop(0, _STRIPE // _CH, _zcopy, 0)
    plsc.subcore_barrier()

    base0 = wid * _EW

    def _chunk(i, carry):
        base = base0 + i * _CH
        pltpu.sync_copy(src_hbm.at[pl.ds(base, _CH)], sidx)
        pltpu.sync_copy(dst_hbm.at[pl.ds(base, _CH)], didx)
        pltpu.sync_copy(c_hbm.at[pl.ds(base, _CH)], cbuf)
        pltpu.sync_copy(w_hbm.at[pl.ds(base, _CH)], wbuf)
        pltpu.async_copy(x1_hbm.at[sidx], rbuf, sem).wait()

        def _mul(g, inner):
            cv16 = cbuf[pl.ds(g * 16, 16)]
            for k in range(16):
                e = g * 16 + k
                cv = cv16[k]
                for j in range(_H // 16):
                    sl = pl.ds(j * 16, 16)
                    rbuf[e, sl] = rbuf[e, sl] * (wbuf[e, sl] * cv)
            return inner

        lax.fori_loop(0, _CH // 16, _mul, 0)
        pltpu.sync_copy(rbuf, acc.at[didx], add=True)
        return carry

    lax.fori_loop(0, _NCH, _chunk, 0)

    plsc.subcore_barrier()
    pltpu.sync_copy(acc.at[pl.ds(sid * _STRIPE, _STRIPE)],
                    out_hbm.at[cid, pl.ds(sid * _STRIPE, _STRIPE)])


def _out_body(p_ref, wl2_ref, bl2_ref, wl3_ref, bl3_ref, o_ref):
    agg = p_ref[0] + p_ref[1]
    x2 = _dot_t(agg, wl2_ref[...]) + bl2_ref[...]
    o_ref[...] = _dot_t(_ssp(x2), wl3_ref[...]) + bl3_ref[...]


_out_call = pl.pallas_call(
    _out_body,
    grid=(5,),
    in_specs=[
        pl.BlockSpec((_NC, _N // 5, _H), lambda i: (0, i, 0)),
        pl.BlockSpec((_H, _F), lambda i: (0, 0)),
        pl.BlockSpec((1, _H), lambda i: (0, 0)),
        pl.BlockSpec((_H, _H), lambda i: (0, 0)),
        pl.BlockSpec((1, _H), lambda i: (0, 0)),
    ],
    out_specs=pl.BlockSpec((_N // 5, _H), lambda i: (i, 0)),
    out_shape=jax.ShapeDtypeStruct((_N, _H), jnp.float32),
)


def kernel(x, ji_pairs, e_ji, e_ji_basis, Wf1, bf1, Wf2, bf2,
           Wl1, Wl2, bl2, Wl3, bl3):
    src = ji_pairs[0].astype(jnp.int32)
    dst = ji_pairs[1].astype(jnp.int32)
    e3d = e_ji.reshape(_NBLK, 1, _BE)

    w_edge, c3d = _filter_call(e_ji_basis, e3d, Wf1, bf1[None, :],
                               Wf2, bf2[None, :])
    x1 = _lin1_call(x, Wl1)
    partial = _sc_aggregate(x1, w_edge, c3d.reshape(_E), src, dst)
    out = _out_call(partial, Wl2, bl2[None, :], Wl3, bl3[None, :])
    return out


# speedup vs baseline: 3.3630x; 1.5335x over previous
import functools
import math

import jax
import jax.numpy as jnp
from jax import lax
from jax.experimental import pallas as pl
from jax.experimental.pallas import tpu as pltpu
from jax.experimental.pallas import tpu_sc as plsc

_N = 10000
_E = 320000
_H = 128
_G = 50
_F = 128
_CUTOFF = 10.0

_BE = 2560
_NBLK = _E // _BE

_NC = 2
_NS = 16
_NW = _NC * _NS
_CH = 64
_NCH = 160
_NPCH = _NCH * _NW
_EP = _NPCH * _CH
_ACC = 10240
_STRIPE = _ACC // _NS

_LOG2 = math.log(2.0)


def _ssp(v):
    return jnp.maximum(v, 0.0) + jnp.log1p(jnp.exp(-jnp.abs(v))) - _LOG2


def _dot_t(a, b):
    return lax.dot_general(a, b, (((1,), (1,)), ((), ())),
                           preferred_element_type=jnp.float32)


def _filter_body(basis_ref, e_ref, wf1_ref, bf1_ref, wf2_ref, bf2_ref,
                 w_ref, c_ref):
    h = _ssp(_dot_t(basis_ref[...], wf1_ref[...]) + bf1_ref[...])
    w_ref[...] = _dot_t(h, wf2_ref[...]) + bf2_ref[...]
    c_ref[...] = 0.25 * (jnp.cos(e_ref[...] * (math.pi / _CUTOFF)) + 1.0)


_filter_call = pl.pallas_call(
    _filter_body,
    grid=(_NBLK,),
    in_specs=[
        pl.BlockSpec((_BE, _G), lambda i: (i, 0)),
        pl.BlockSpec((1, 1, _BE), lambda i: (i, 0, 0)),
        pl.BlockSpec((_F, _G), lambda i: (0, 0)),
        pl.BlockSpec((1, _F), lambda i: (0, 0)),
        pl.BlockSpec((_F, _F), lambda i: (0, 0)),
        pl.BlockSpec((1, _F), lambda i: (0, 0)),
    ],
    out_specs=[
        pl.BlockSpec((_BE, _F), lambda i: (i, 0)),
        pl.BlockSpec((1, 1, _BE), lambda i: (i, 0, 0)),
    ],
    out_shape=[
        jax.ShapeDtypeStruct((_E, _F), jnp.float32),
        jax.ShapeDtypeStruct((_NBLK, 1, _BE), jnp.float32),
    ],
)


def _lin1_body(x_ref, wl1_ref, o_ref):
    o_ref[...] = _dot_t(x_ref[...], wl1_ref[...])


_lin1_call = pl.pallas_call(
    _lin1_body,
    grid=(5,),
    in_specs=[
        pl.BlockSpec((_N // 5, _H), lambda i: (i, 0)),
        pl.BlockSpec((_F, _H), lambda i: (0, 0)),
    ],
    out_specs=pl.BlockSpec((_N // 5, _F), lambda i: (i, 0)),
    out_shape=jax.ShapeDtypeStruct((_N, _F), jnp.float32),
)


_sc_mesh = plsc.VectorSubcoreMesh(core_axis_name="c", subcore_axis_name="s")


@functools.partial(
    pl.kernel,
    mesh=_sc_mesh,
    out_type=jax.ShapeDtypeStruct((_NC, _ACC, _H), jnp.float32),
    scratch_types=[
        pltpu.VMEM((4, _CH), jnp.int32),
        pltpu.VMEM((4, _CH), jnp.int32),
        pltpu.VMEM((4, _CH), jnp.float32),
        pltpu.VMEM((2, _CH, _H), jnp.float32),
        pltpu.VMEM((2, _CH, _H), jnp.float32),
        pltpu.VMEM_SHARED((_ACC, _H), jnp.float32),
        pltpu.SemaphoreType.DMA,
        pltpu.SemaphoreType.DMA,
        pltpu.SemaphoreType.DMA,
        pltpu.SemaphoreType.DMA,
        pltpu.SemaphoreType.DMA,
        pltpu.SemaphoreType.DMA,
        pltpu.SemaphoreType.DMA,
        pltpu.SemaphoreType.DMA,
    ],
)
def _sc_aggregate(x1_hbm, w_hbm, c_hbm, src_hbm, dst_hbm, out_hbm,
                  sidx, didx, cbuf, wbuf, rbuf, acc,
                  i0s, i1s, i2s, i3s, g0s, g1s, w0s, w1s):
    cid = lax.axis_index("c")
    sid = lax.axis_index("s")
    wid = cid * _NS + sid
    isem = (i0s, i1s, i2s, i3s)
    gsem = (g0s, g1s)
    wsem = (w0s, w1s)

    def _zrow(e, carry):
        for j in range(_H // 16):
            rbuf[0, e, pl.ds(j * 16, 16)] = jnp.zeros((16,), jnp.float32)
        return carry

    lax.fori_loop(0, _CH, _zrow, 0)

    def _zcopy(k, carry):
        pltpu.sync_copy(rbuf.at[0], acc.at[pl.ds(sid * _STRIPE + k * _CH, _CH)])
        return carry

    lax.fori_loop(0, _STRIPE // _CH, _zcopy, 0)
    plsc.subcore_barrier()

    base0 = wid * _NCH

    def _iload(i, a):
        e0 = (base0 + i) * _CH
        pltpu.async_copy(src_hbm.at[pl.ds(e0, _CH)], sidx.at[a], isem[a])
        pltpu.async_copy(dst_hbm.at[pl.ds(e0, _CH)], didx.at[a], isem[a])
        pltpu.async_copy(c_hbm.at[pl.ds(e0, _CH)], cbuf.at[a], isem[a])

    def _iwait(i, a):
        e0 = (base0 + i) * _CH
        pltpu.make_async_copy(src_hbm.at[pl.ds(e0, _CH)], sidx.at[a],
                              isem[a]).wait()
        pltpu.make_async_copy(dst_hbm.at[pl.ds(e0, _CH)], didx.at[a],
                              isem[a]).wait()
        pltpu.make_async_copy(c_hbm.at[pl.ds(e0, _CH)], cbuf.at[a],
                              isem[a]).wait()

    def _wslab(i):
        return jnp.minimum((base0 + i) * _CH, _E - _CH)

    def _start(i, a, b):
        pltpu.async_copy(w_hbm.at[pl.ds(_wslab(i), _CH)], wbuf.at[b], wsem[b])
        pltpu.async_copy(x1_hbm.at[sidx.at[a]], rbuf.at[b], gsem[b])

    def _finish(i, a, b):
        pltpu.make_async_copy(w_hbm.at[pl.ds(_wslab(i), _CH)], wbuf.at[b],
                              wsem[b]).wait()
        pltpu.make_async_copy(x1_hbm.at[sidx.at[a]], rbuf.at[b],
                              gsem[b]).wait()

        def _mul(g, inner):
            cv16 = cbuf[a, pl.ds(g * 16, 16)]
            for k in range(16):
                e = g * 16 + k
                cv = cv16[k]
                for j in range(_H // 16):
                    sl = pl.ds(j * 16, 16)
                    rbuf[b, e, sl] = rbuf[b, e, sl] * (wbuf[b, e, sl] * cv)
            return inner

        lax.fori_loop(0, _CH // 16, _mul, 0)
        pltpu.sync_copy(rbuf.at[b], acc.at[didx.at[a]], add=True)

    _iload(0, 0)
    _iload(1, 1)
    _iwait(0, 0)
    _start(0, 0, 0)

    def _group(g4, carry):
        for k in range(4):
            i = g4 * 4 + k

            @pl.when(i < _NCH - 1)
            def _adv():
                _iwait(i + 1, (k + 1) % 4)
                _start(i + 1, (k + 1) % 4, (k + 1) % 2)

            @pl.when(i < _NCH - 2)
            def _pref():
                _iload(i + 2, (k + 2) % 4)

            _finish(i, k % 4, k % 2)
        return carry

    lax.fori_loop(0, _NCH // 4, _group, 0)

    plsc.subcore_barrier()
    pltpu.sync_copy(acc.at[pl.ds(sid * _STRIPE, _STRIPE)],
                    out_hbm.at[cid, pl.ds(sid * _STRIPE, _STRIPE)])


def _out_body(p_ref, wl2_ref, bl2_ref, wl3_ref, bl3_ref, o_ref):
    agg = p_ref[0] + p_ref[1]
    x2 = _dot_t(agg, wl2_ref[...]) + bl2_ref[...]
    o_ref[...] = _dot_t(_ssp(x2), wl3_ref[...]) + bl3_ref[...]


_out_call = pl.pallas_call(
    _out_body,
    grid=(5,),
    in_specs=[
        pl.BlockSpec((_NC, _N // 5, _H), lambda i: (0, i, 0)),
        pl.BlockSpec((_H, _F), lambda i: (0, 0)),
        pl.BlockSpec((1, _H), lambda i: (0, 0)),
        pl.BlockSpec((_H, _H), lambda i: (0, 0)),
        pl.BlockSpec((1, _H), lambda i: (0, 0)),
    ],
    out_specs=pl.BlockSpec((_N // 5, _H), lambda i: (i, 0)),
    out_shape=jax.ShapeDtypeStruct((_N, _H), jnp.float32),
)


def kernel(x, ji_pairs, e_ji, e_ji_basis, Wf1, bf1, Wf2, bf2,
           Wl1, Wl2, bl2, Wl3, bl3):
    npad = _EP - _E
    fill = jnp.arange(npad, dtype=jnp.int32) % _N
    src = jnp.concatenate([ji_pairs[0].astype(jnp.int32), fill])
    dst = jnp.concatenate([ji_pairs[1].astype(jnp.int32), fill])
    e3d = e_ji.reshape(_NBLK, 1, _BE)

    w_edge, c3d = _filter_call(e_ji_basis, e3d, Wf1, bf1[None, :],
                               Wf2, bf2[None, :])
    x1 = _lin1_call(x, Wl1)
    c_pad = jnp.concatenate([c3d.reshape(_E), jnp.zeros(npad, jnp.float32)])
    partial = _sc_aggregate(x1, w_edge, c_pad, src, dst)
    out = _out_call(partial, Wl2, bl2[None, :], Wl3, bl3[None, :])
    return out
